# baseline (speedup 1.0000x reference)
import functools

import jax
import jax.numpy as jnp
from jax import lax
from jax.experimental import pallas as pl
from jax.experimental.pallas import tpu as pltpu
from jax.experimental.pallas import tpu_sc as plsc

_NC = 2
_NS = 16
_NW = _NC * _NS
_LANES = 16

_CHUNK = 320
_PDIM = 128
_SEGS = [(0, 128), (128, 128), (256, 64)]
assert sum(s[1] for s in _SEGS) == _CHUNK


def _make_embed(num_tok, seq_len, vocab):
    assert num_tok % (_NW * _CHUNK) == 0
    b_per_w = num_tok // _NW
    n_chunks = b_per_w // _CHUNK
    assert n_chunks % 2 == 0 and n_chunks >= 4
    mesh = plsc.VectorSubcoreMesh(core_axis_name="c", subcore_axis_name="s")

    @functools.partial(
        pl.kernel,
        mesh=mesh,
        out_type=jax.ShapeDtypeStruct((num_tok, _PDIM), jnp.float32),
        scratch_types=[
            pltpu.VMEM((_CHUNK,), jnp.int32),
            pltpu.VMEM((_CHUNK,), jnp.int32),
            pltpu.VMEM((3, 128), jnp.int32),
            pltpu.VMEM((3, 128), jnp.int32),
            pltpu.VMEM((_CHUNK, _PDIM), jnp.float32),
            pltpu.VMEM((_CHUNK, _PDIM), jnp.float32),
            pltpu.VMEM((_LANES,), jnp.int32),
            pltpu.SemaphoreType.DMA,
            pltpu.SemaphoreType.DMA,
            pltpu.SemaphoreType.DMA,
            pltpu.SemaphoreType.DMA,
            pltpu.SemaphoreType.DMA,
            pltpu.SemaphoreType.DMA,
        ],
    )
    def embed(text_hbm, msl_hbm, table_hbm, out_hbm, tok0_v, tok1_v,
              idx0_v, idx1_v, rows0_v, rows1_v, msl_v,
              tsem0, tsem1, gsem0, gsem1, osem0, osem1):
        wid = lax.axis_index("s") * _NC + lax.axis_index("c")
        gbase = wid * b_per_w
        tok_v = (tok0_v, tok1_v)
        idx_v = (idx0_v, idx1_v)
        rows_v = (rows0_v, rows1_v)
        tsem = (tsem0, tsem1)
        gsem = (gsem0, gsem1)
        osem = (osem0, osem1)
        pltpu.sync_copy(msl_hbm, msl_v)
        msl_vec = msl_v[...]
        lane = lax.iota(jnp.int32, _LANES)

        def tok_load(j, b):
            return pltpu.async_copy(
                text_hbm.at[pl.ds(gbase + j * _CHUNK, _CHUNK)],
                tok_v[b], tsem[b])

        def compute_idx(j, b):
            for g in range(_CHUNK // _LANES):
                o = g * _LANES
                col = lax.rem(lane + (j * _CHUNK + o), seq_len)
                tok = tok_v[b][pl.ds(o, _LANES)]
                idx_v[b][o // 128, pl.ds(o % 128, _LANES)] = jnp.where(
                    col < msl_vec, tok + 1, 0)

        def gather_start(j, b):
            for s, (o, ln) in enumerate(_SEGS):
                pltpu.async_copy(
                    table_hbm.at[idx_v[b].at[s, pl.ds(0, ln)]],
                    rows_v[b].at[pl.ds(o, ln)], gsem[b])

        def gather_wait(b):
            for s, (o, ln) in enumerate(_SEGS):
                pltpu.make_async_copy(
                    table_hbm.at[idx_v[b].at[s, pl.ds(0, ln)]],
                    rows_v[b].at[pl.ds(o, ln)], gsem[b]).wait()

        def out_start(j, b):
            pltpu.async_copy(
                rows_v[b], out_hbm.at[pl.ds(gbase + j * _CHUNK, _CHUNK)],
                osem[b])

        def out_wait(j, b):
            pltpu.make_async_copy(
                rows_v[b], out_hbm.at[pl.ds(gbase + j * _CHUNK, _CHUNK)],
                osem[b]).wait()

        tok_load(0, 0).wait()
        tok_load(1, 1)
        compute_idx(0, 0)
        gather_start(0, 0)
        tok_load(2, 0)
        pltpu.make_async_copy(
            text_hbm.at[pl.ds(0, _CHUNK)], tok_v[1], tsem[1]).wait()
        compute_idx(1, 1)
        gather_start(1, 1)
        tok_load(3, 1)
        gather_wait(0)
        out_start(0, 0)
        pltpu.make_async_copy(
            text_hbm.at[pl.ds(0, _CHUNK)], tok_v[0], tsem[0]).wait()

        def pair_body(j0, carry):
            for b in (0, 1):
                j = 2 * j0 + b
                compute_idx(j, b)
                out_wait(j - 2, b)
                gather_start(j, b)
                tok_load(j + 2, b)
                gather_wait(1 - b)
                out_start(j - 1, 1 - b)
                pltpu.make_async_copy(
                    text_hbm.at[pl.ds(0, _CHUNK)], tok_v[1 - b],
                    tsem[1 - b]).wait()
            return carry

        lax.fori_loop(1, n_chunks // 2 - 1, pair_body, 0)

        j = n_chunks - 2
        compute_idx(j, 0)
        out_wait(j - 2, 0)
        gather_start(j, 0)
        gather_wait(1)
        out_start(j - 1, 1)
        pltpu.make_async_copy(
            text_hbm.at[pl.ds(0, _CHUNK)], tok_v[1], tsem[1]).wait()
        j = n_chunks - 1
        compute_idx(j, 1)
        out_wait(j - 2, 1)
        gather_start(j, 1)
        gather_wait(0)
        out_start(j - 1, 0)
        gather_wait(1)
        out_wait(j - 1, 0)
        out_start(j, 1)
        out_wait(j, 1)

    return embed


def kernel(text, max_seq_len, embed_table):
    bsz, seq_len = text.shape
    vocab, dim = embed_table.shape
    num_tok = bsz * seq_len
    text_flat = text.reshape(num_tok)
    table_pad = jnp.pad(embed_table, ((0, 0), (0, _PDIM - dim)))
    msl = jnp.full((_LANES,), max_seq_len, dtype=jnp.int32)
    out = _make_embed(num_tok, seq_len, vocab)(text_flat, msl, table_pad)
    return out[:, :dim].reshape(bsz, seq_len, dim)
